# baseline (speedup 1.0000x reference)
import functools

import jax
import jax.numpy as jnp
from jax.experimental import pallas as pl

RADIUS = 0.2
N = 1024
CT = 256
OUT_CH = 128


def _tc_kernel(xyzT_ref, xyzS_ref, W1_ref, W2_ref, W3_ref,
               b1_ref, b2_ref, b3_ref, out_ref):
    ct = pl.program_id(1)
    judge = jnp.float32(RADIUS * RADIUS)

    W21 = jnp.dot(W2_ref[...], W1_ref[...], preferred_element_type=jnp.float32, precision=jax.lax.Precision.HIGHEST)
    W321 = jnp.dot(W3_ref[...], W21, preferred_element_type=jnp.float32, precision=jax.lax.Precision.HIGHEST)
    be = jax.lax.dot_general(b1_ref[...], W2_ref[...],
                             (((1,), (1,)), ((), ())), precision=jax.lax.Precision.HIGHEST) + b2_ref[...]
    beff = jax.lax.dot_general(be, W3_ref[...],
                               (((1,), (1,)), ((), ())), precision=jax.lax.Precision.HIGHEST) + b3_ref[...]

    xyz_all = xyzS_ref[0]
    P = jax.lax.dot_general(xyz_all, W321, (((1,), (1,)), ((), ())),
                            preferred_element_type=jnp.float32, precision=jax.lax.Precision.HIGHEST)
    P_hi = P.astype(jnp.bfloat16)
    P_lo = (P - P_hi.astype(jnp.float32)).astype(jnp.bfloat16)

    xs = xyzT_ref[0, 0:1, :]
    ys = xyzT_ref[0, 1:2, :]
    zs = xyzT_ref[0, 2:3, :]
    c3 = xyzS_ref[0, pl.ds(ct * CT, CT), :]
    xc = c3[:, 0:1]
    yc = c3[:, 1:2]
    zc = c3[:, 2:3]

    dx = xs - xc
    dy = ys - yc
    dz = zs - zc
    dist = dx * dx + dy * dy + dz * dz

    one = jnp.float32(1.0)
    i0 = jnp.int32(0)
    neg_eps = jnp.float32(-(2.0 ** -25))
    s = (jnp.where(dx >= neg_eps, jnp.int32(4), i0)
         + jnp.where(dy >= neg_eps, jnp.int32(2), i0)
         + jnp.where(dz >= neg_eps, jnp.int32(1), i0))
    valid = (dist > jnp.float32(1e-10)) & (dist < judge)

    lane = jax.lax.broadcasted_iota(jnp.int32, (CT, N), 1)
    row = jax.lax.broadcasted_iota(jnp.int32, (CT, N), 0)
    eye = lane == (row + ct * CT)
    base = jnp.where(eye, judge, jnp.float32(1e10))
    dv = jnp.where(valid, dist, base)

    Pc = jax.lax.dot_general(c3, W321, (((1,), (1,)), ((), ())),
                             preferred_element_type=jnp.float32, precision=jax.lax.Precision.HIGHEST)
    offs = beff - Pc

    ones8 = jnp.ones((N, 8), jnp.bfloat16)
    g = []
    for i in range(8):
        d_i = jnp.where(s == i, dv, base)
        mn = jnp.min(d_i, axis=1, keepdims=True)
        ebf = (d_i == mn).astype(jnp.bfloat16)
        cnt = jnp.dot(ebf, ones8, preferred_element_type=jnp.float32)[:, 0:1]
        rec = jnp.where(cnt > jnp.float32(1.5), jnp.float32(0.5), one)
        Gi = (jnp.dot(ebf, P_hi, preferred_element_type=jnp.float32)
              + jnp.dot(ebf, P_lo, preferred_element_type=jnp.float32))
        g.append(Gi * rec + offs)

    m01 = jnp.maximum(g[0], g[1])
    m23 = jnp.maximum(g[2], g[3])
    m45 = jnp.maximum(g[4], g[5])
    m67 = jnp.maximum(g[6], g[7])
    q0 = jnp.maximum(m01, m23)
    q1 = jnp.maximum(m45, m67)

    C = OUT_CH
    cols = [q0, q1, m01, m23, m45, m67] + g
    for s, v in enumerate(cols):
        out_ref[0, :, s * C:(s + 1) * C] = v


@jax.jit
def kernel(x, W1, b1, W2, b2, W3, b3):
    B, t, n, c = x.shape
    BT = B * t
    xyzS = x.reshape(BT, n, c)
    xyzT = jnp.transpose(xyzS, (0, 2, 1))
    b1r = b1.reshape(1, OUT_CH)
    b2r = b2.reshape(1, OUT_CH)
    b3r = b3.reshape(1, OUT_CH)

    out = pl.pallas_call(
        _tc_kernel,
        grid=(BT, n // CT),
        in_specs=[
            pl.BlockSpec((1, 3, n), lambda b, ct: (b, 0, 0)),
            pl.BlockSpec((1, n, 3), lambda b, ct: (b, 0, 0)),
            pl.BlockSpec((OUT_CH, 3), lambda b, ct: (0, 0)),
            pl.BlockSpec((OUT_CH, OUT_CH), lambda b, ct: (0, 0)),
            pl.BlockSpec((OUT_CH, OUT_CH), lambda b, ct: (0, 0)),
            pl.BlockSpec((1, OUT_CH), lambda b, ct: (0, 0)),
            pl.BlockSpec((1, OUT_CH), lambda b, ct: (0, 0)),
            pl.BlockSpec((1, OUT_CH), lambda b, ct: (0, 0)),
        ],
        out_specs=pl.BlockSpec((1, CT, 14 * OUT_CH), lambda b, ct: (b, ct, 0)),
        out_shape=jax.ShapeDtypeStruct((BT, n, 14 * OUT_CH), jnp.float32),
    )(xyzT, xyzS, W1, W2, W3, b1r, b2r, b3r)
    return out.reshape(B, t, n, 14 * OUT_CH)


# speedup vs baseline: 1.0597x; 1.0597x over previous
import functools

import jax
import jax.numpy as jnp
from jax.experimental import pallas as pl
from jax.experimental.pallas import tpu as pltpu

RADIUS = 0.2
N = 1024
CT = 256
OUT_CH = 128


def _tc_kernel(xyzT_ref, xyzS_ref, W1_ref, W2_ref, W3_ref,
               b1_ref, b2_ref, b3_ref, out_ref):
    ct = pl.program_id(1)
    judge = jnp.float32(RADIUS * RADIUS)

    W21 = jnp.dot(W2_ref[...], W1_ref[...], preferred_element_type=jnp.float32, precision=jax.lax.Precision.HIGHEST)
    W321 = jnp.dot(W3_ref[...], W21, preferred_element_type=jnp.float32, precision=jax.lax.Precision.HIGHEST)
    be = jax.lax.dot_general(b1_ref[...], W2_ref[...],
                             (((1,), (1,)), ((), ())), precision=jax.lax.Precision.HIGHEST) + b2_ref[...]
    beff = jax.lax.dot_general(be, W3_ref[...],
                               (((1,), (1,)), ((), ())), precision=jax.lax.Precision.HIGHEST) + b3_ref[...]

    xyz_all = xyzS_ref[0]
    P = jax.lax.dot_general(xyz_all, W321, (((1,), (1,)), ((), ())),
                            preferred_element_type=jnp.float32, precision=jax.lax.Precision.HIGHEST)
    P_hi = P.astype(jnp.bfloat16)
    P_lo = (P - P_hi.astype(jnp.float32)).astype(jnp.bfloat16)

    xs = xyzT_ref[0, 0:1, :]
    ys = xyzT_ref[0, 1:2, :]
    zs = xyzT_ref[0, 2:3, :]
    c3 = xyzS_ref[0, pl.ds(ct * CT, CT), :]
    xc = c3[:, 0:1]
    yc = c3[:, 1:2]
    zc = c3[:, 2:3]

    dx = xs - xc
    dy = ys - yc
    dz = zs - zc
    dist = dx * dx + dy * dy + dz * dz

    one = jnp.float32(1.0)
    i0 = jnp.int32(0)
    neg_eps = jnp.float32(-(2.0 ** -25))
    s = (jnp.where(dx >= neg_eps, jnp.int32(4), i0)
         + jnp.where(dy >= neg_eps, jnp.int32(2), i0)
         + jnp.where(dz >= neg_eps, jnp.int32(1), i0))
    valid = (dist > jnp.float32(1e-10)) & (dist < judge)

    lane = jax.lax.broadcasted_iota(jnp.int32, (CT, N), 1)
    row = jax.lax.broadcasted_iota(jnp.int32, (CT, N), 0)
    eye = lane == (row + ct * CT)
    base = jnp.where(eye, judge, jnp.float32(1e10))
    dv = jnp.where(valid, dist, base)

    Pc = jax.lax.dot_general(c3, W321, (((1,), (1,)), ((), ())),
                             preferred_element_type=jnp.float32, precision=jax.lax.Precision.HIGHEST)
    offs = beff - Pc

    g = []
    for i in range(8):
        d_i = jnp.where(s == i, dv, base)
        mn = jnp.min(d_i, axis=1, keepdims=True)
        idx = jnp.min(jnp.where(d_i == mn, lane, N), axis=1, keepdims=True)
        onehot = (lane == idx).astype(jnp.bfloat16)
        Gi = (jnp.dot(onehot, P_hi, preferred_element_type=jnp.float32)
              + jnp.dot(onehot, P_lo, preferred_element_type=jnp.float32))
        g.append(Gi + offs)

    m01 = jnp.maximum(g[0], g[1])
    m23 = jnp.maximum(g[2], g[3])
    m45 = jnp.maximum(g[4], g[5])
    m67 = jnp.maximum(g[6], g[7])
    q0 = jnp.maximum(m01, m23)
    q1 = jnp.maximum(m45, m67)

    C = OUT_CH
    cols = [q0, q1, m01, m23, m45, m67] + g
    for s, v in enumerate(cols):
        out_ref[0, :, s * C:(s + 1) * C] = v


@jax.jit
def kernel(x, W1, b1, W2, b2, W3, b3):
    B, t, n, c = x.shape
    BT = B * t
    xyzS = x.reshape(BT, n, c)
    xyzT = jnp.transpose(xyzS, (0, 2, 1))
    b1r = b1.reshape(1, OUT_CH)
    b2r = b2.reshape(1, OUT_CH)
    b3r = b3.reshape(1, OUT_CH)

    out = pl.pallas_call(
        _tc_kernel,
        grid=(BT, n // CT),
        in_specs=[
            pl.BlockSpec((1, 3, n), lambda b, ct: (b, 0, 0)),
            pl.BlockSpec((1, n, 3), lambda b, ct: (b, 0, 0)),
            pl.BlockSpec((OUT_CH, 3), lambda b, ct: (0, 0)),
            pl.BlockSpec((OUT_CH, OUT_CH), lambda b, ct: (0, 0)),
            pl.BlockSpec((OUT_CH, OUT_CH), lambda b, ct: (0, 0)),
            pl.BlockSpec((1, OUT_CH), lambda b, ct: (0, 0)),
            pl.BlockSpec((1, OUT_CH), lambda b, ct: (0, 0)),
            pl.BlockSpec((1, OUT_CH), lambda b, ct: (0, 0)),
        ],
        out_specs=pl.BlockSpec((1, CT, 14 * OUT_CH), lambda b, ct: (b, ct, 0)),
        out_shape=jax.ShapeDtypeStruct((BT, n, 14 * OUT_CH), jnp.float32),
        compiler_params=pltpu.CompilerParams(
            dimension_semantics=("parallel", "parallel")),
    )(xyzT, xyzS, W1, W2, W3, b1r, b2r, b3r)
    return out.reshape(B, t, n, 14 * OUT_CH)


# speedup vs baseline: 1.2119x; 1.1436x over previous
import functools

import jax
import jax.numpy as jnp
from jax.experimental import pallas as pl
from jax.experimental.pallas import tpu as pltpu

RADIUS = 0.2
N = 1024
CT = 512
OUT_CH = 128


def _tc_kernel(xyzT_ref, xyzS_ref, W1_ref, W2_ref, W3_ref,
               b1_ref, b2_ref, b3_ref, out_ref):
    ct = pl.program_id(1)
    judge = jnp.float32(RADIUS * RADIUS)

    W21 = jnp.dot(W2_ref[...], W1_ref[...], preferred_element_type=jnp.float32, precision=jax.lax.Precision.HIGHEST)
    W321 = jnp.dot(W3_ref[...], W21, preferred_element_type=jnp.float32, precision=jax.lax.Precision.HIGHEST)
    be = jax.lax.dot_general(b1_ref[...], W2_ref[...],
                             (((1,), (1,)), ((), ())), precision=jax.lax.Precision.HIGHEST) + b2_ref[...]
    beff = jax.lax.dot_general(be, W3_ref[...],
                               (((1,), (1,)), ((), ())), precision=jax.lax.Precision.HIGHEST) + b3_ref[...]

    xyz_all = xyzS_ref[0]
    P = jax.lax.dot_general(xyz_all, W321, (((1,), (1,)), ((), ())),
                            preferred_element_type=jnp.float32, precision=jax.lax.Precision.HIGHEST)
    P_hi = P.astype(jnp.bfloat16)
    P_lo = (P - P_hi.astype(jnp.float32)).astype(jnp.bfloat16)

    xs = xyzT_ref[0, 0:1, :]
    ys = xyzT_ref[0, 1:2, :]
    zs = xyzT_ref[0, 2:3, :]
    c3 = xyzS_ref[0, pl.ds(ct * CT, CT), :]
    xc = c3[:, 0:1]
    yc = c3[:, 1:2]
    zc = c3[:, 2:3]

    dx = xs - xc
    dy = ys - yc
    dz = zs - zc
    dist = dx * dx + dy * dy + dz * dz

    one = jnp.float32(1.0)
    i0 = jnp.int32(0)
    neg_eps = jnp.float32(-(2.0 ** -25))
    s = (jnp.where(dx >= neg_eps, jnp.int32(4), i0)
         + jnp.where(dy >= neg_eps, jnp.int32(2), i0)
         + jnp.where(dz >= neg_eps, jnp.int32(1), i0))
    valid = (dist > jnp.float32(1e-10)) & (dist < judge)

    lane = jax.lax.broadcasted_iota(jnp.int32, (CT, N), 1)
    row = jax.lax.broadcasted_iota(jnp.int32, (CT, N), 0)
    eye = lane == (row + ct * CT)
    base = jnp.where(eye, judge, jnp.float32(1e10))
    dv = jnp.where(valid, dist, base)

    Pc = jax.lax.dot_general(c3, W321, (((1,), (1,)), ((), ())),
                             preferred_element_type=jnp.float32, precision=jax.lax.Precision.HIGHEST)
    offs = beff - Pc

    g = []
    for i in range(8):
        d_i = jnp.where(s == i, dv, base)
        mn = jnp.min(d_i, axis=1, keepdims=True)
        idx = jnp.min(jnp.where(d_i == mn, lane, N), axis=1, keepdims=True)
        onehot = (lane == idx).astype(jnp.bfloat16)
        Gi = (jnp.dot(onehot, P_hi, preferred_element_type=jnp.float32)
              + jnp.dot(onehot, P_lo, preferred_element_type=jnp.float32))
        g.append(Gi + offs)

    m01 = jnp.maximum(g[0], g[1])
    m23 = jnp.maximum(g[2], g[3])
    m45 = jnp.maximum(g[4], g[5])
    m67 = jnp.maximum(g[6], g[7])
    q0 = jnp.maximum(m01, m23)
    q1 = jnp.maximum(m45, m67)

    C = OUT_CH
    cols = [q0, q1, m01, m23, m45, m67] + g
    for s, v in enumerate(cols):
        out_ref[0, :, s * C:(s + 1) * C] = v


@jax.jit
def kernel(x, W1, b1, W2, b2, W3, b3):
    B, t, n, c = x.shape
    BT = B * t
    xyzS = x.reshape(BT, n, c)
    xyzT = jnp.transpose(xyzS, (0, 2, 1))
    b1r = b1.reshape(1, OUT_CH)
    b2r = b2.reshape(1, OUT_CH)
    b3r = b3.reshape(1, OUT_CH)

    out = pl.pallas_call(
        _tc_kernel,
        grid=(BT, n // CT),
        in_specs=[
            pl.BlockSpec((1, 3, n), lambda b, ct: (b, 0, 0)),
            pl.BlockSpec((1, n, 3), lambda b, ct: (b, 0, 0)),
            pl.BlockSpec((OUT_CH, 3), lambda b, ct: (0, 0)),
            pl.BlockSpec((OUT_CH, OUT_CH), lambda b, ct: (0, 0)),
            pl.BlockSpec((OUT_CH, OUT_CH), lambda b, ct: (0, 0)),
            pl.BlockSpec((1, OUT_CH), lambda b, ct: (0, 0)),
            pl.BlockSpec((1, OUT_CH), lambda b, ct: (0, 0)),
            pl.BlockSpec((1, OUT_CH), lambda b, ct: (0, 0)),
        ],
        out_specs=pl.BlockSpec((1, CT, 14 * OUT_CH), lambda b, ct: (b, ct, 0)),
        out_shape=jax.ShapeDtypeStruct((BT, n, 14 * OUT_CH), jnp.float32),
        compiler_params=pltpu.CompilerParams(
            dimension_semantics=("parallel", "parallel")),
    )(xyzT, xyzS, W1, W2, W3, b1r, b2r, b3r)
    return out.reshape(B, t, n, 14 * OUT_CH)


# speedup vs baseline: 1.2273x; 1.0127x over previous
import functools

import jax
import jax.numpy as jnp
from jax.experimental import pallas as pl
from jax.experimental.pallas import tpu as pltpu

RADIUS = 0.2
N = 1024
CT = 1024
OUT_CH = 128


def _tc_kernel(xyzT_ref, xyzS_ref, W1_ref, W2_ref, W3_ref,
               b1_ref, b2_ref, b3_ref, out_ref):
    ct = pl.program_id(1)
    judge = jnp.float32(RADIUS * RADIUS)

    W21 = jnp.dot(W2_ref[...], W1_ref[...], preferred_element_type=jnp.float32, precision=jax.lax.Precision.HIGHEST)
    W321 = jnp.dot(W3_ref[...], W21, preferred_element_type=jnp.float32, precision=jax.lax.Precision.HIGHEST)
    be = jax.lax.dot_general(b1_ref[...], W2_ref[...],
                             (((1,), (1,)), ((), ())), precision=jax.lax.Precision.HIGHEST) + b2_ref[...]
    beff = jax.lax.dot_general(be, W3_ref[...],
                               (((1,), (1,)), ((), ())), precision=jax.lax.Precision.HIGHEST) + b3_ref[...]

    xyz_all = xyzS_ref[0]
    P = jax.lax.dot_general(xyz_all, W321, (((1,), (1,)), ((), ())),
                            preferred_element_type=jnp.float32, precision=jax.lax.Precision.HIGHEST)
    P_hi = P.astype(jnp.bfloat16)
    P_lo = (P - P_hi.astype(jnp.float32)).astype(jnp.bfloat16)

    xs = xyzT_ref[0, 0:1, :]
    ys = xyzT_ref[0, 1:2, :]
    zs = xyzT_ref[0, 2:3, :]
    c3 = xyzS_ref[0, pl.ds(ct * CT, CT), :]
    xc = c3[:, 0:1]
    yc = c3[:, 1:2]
    zc = c3[:, 2:3]

    dx = xs - xc
    dy = ys - yc
    dz = zs - zc
    dist = dx * dx + dy * dy + dz * dz

    one = jnp.float32(1.0)
    i0 = jnp.int32(0)
    neg_eps = jnp.float32(-(2.0 ** -25))
    s = (jnp.where(dx >= neg_eps, jnp.int32(4), i0)
         + jnp.where(dy >= neg_eps, jnp.int32(2), i0)
         + jnp.where(dz >= neg_eps, jnp.int32(1), i0))
    valid = (dist > jnp.float32(1e-10)) & (dist < judge)

    lane = jax.lax.broadcasted_iota(jnp.int32, (CT, N), 1)
    row = jax.lax.broadcasted_iota(jnp.int32, (CT, N), 0)
    eye = lane == (row + ct * CT)
    base = jnp.where(eye, judge, jnp.float32(1e10))
    dv = jnp.where(valid, dist, base)

    Pc = jax.lax.dot_general(c3, W321, (((1,), (1,)), ((), ())),
                             preferred_element_type=jnp.float32, precision=jax.lax.Precision.HIGHEST)
    offs = beff - Pc

    g = []
    for i in range(8):
        d_i = jnp.where(s == i, dv, base)
        mn = jnp.min(d_i, axis=1, keepdims=True)
        idx = jnp.min(jnp.where(d_i == mn, lane, N), axis=1, keepdims=True)
        onehot = (lane == idx).astype(jnp.bfloat16)
        Gi = (jnp.dot(onehot, P_hi, preferred_element_type=jnp.float32)
              + jnp.dot(onehot, P_lo, preferred_element_type=jnp.float32))
        g.append(Gi + offs)

    m01 = jnp.maximum(g[0], g[1])
    m23 = jnp.maximum(g[2], g[3])
    m45 = jnp.maximum(g[4], g[5])
    m67 = jnp.maximum(g[6], g[7])
    q0 = jnp.maximum(m01, m23)
    q1 = jnp.maximum(m45, m67)

    C = OUT_CH
    cols = [q0, q1, m01, m23, m45, m67] + g
    for s, v in enumerate(cols):
        out_ref[0, :, s * C:(s + 1) * C] = v


@jax.jit
def kernel(x, W1, b1, W2, b2, W3, b3):
    B, t, n, c = x.shape
    BT = B * t
    xyzS = x.reshape(BT, n, c)
    xyzT = jnp.transpose(xyzS, (0, 2, 1))
    b1r = b1.reshape(1, OUT_CH)
    b2r = b2.reshape(1, OUT_CH)
    b3r = b3.reshape(1, OUT_CH)

    out = pl.pallas_call(
        _tc_kernel,
        grid=(BT, n // CT),
        in_specs=[
            pl.BlockSpec((1, 3, n), lambda b, ct: (b, 0, 0)),
            pl.BlockSpec((1, n, 3), lambda b, ct: (b, 0, 0)),
            pl.BlockSpec((OUT_CH, 3), lambda b, ct: (0, 0)),
            pl.BlockSpec((OUT_CH, OUT_CH), lambda b, ct: (0, 0)),
            pl.BlockSpec((OUT_CH, OUT_CH), lambda b, ct: (0, 0)),
            pl.BlockSpec((1, OUT_CH), lambda b, ct: (0, 0)),
            pl.BlockSpec((1, OUT_CH), lambda b, ct: (0, 0)),
            pl.BlockSpec((1, OUT_CH), lambda b, ct: (0, 0)),
        ],
        out_specs=pl.BlockSpec((1, CT, 14 * OUT_CH), lambda b, ct: (b, ct, 0)),
        out_shape=jax.ShapeDtypeStruct((BT, n, 14 * OUT_CH), jnp.float32),
        compiler_params=pltpu.CompilerParams(
            dimension_semantics=("parallel", "parallel")),
    )(xyzT, xyzS, W1, W2, W3, b1r, b2r, b3r)
    return out.reshape(B, t, n, 14 * OUT_CH)
